# baseline (device time: 8565 ns/iter reference)
import jax
import jax.numpy as jnp
from jax import lax
from jax.experimental import pallas as pl
from jax.experimental.pallas import tpu as pltpu

N_DEV = 8
H = 2


def kernel(x):
    m_per, n = x.shape
    cw = n // H

    def body(x_ref, out_ref, part_buf, send_buf, send_sems, recv_sems):
        my_pos = lax.axis_index("i")

        barrier_sem = pltpu.get_barrier_semaphore()
        for j in range(1, N_DEV):
            pl.semaphore_signal(
                barrier_sem, inc=1,
                device_id=(lax.rem(my_pos + j, N_DEV),),
                device_id_type=pl.DeviceIdType.MESH,
            )

        rows = lax.broadcasted_iota(jnp.int32, (m_per, cw), 0)
        locals_, rdmas = [], []
        for h in range(H):
            c0 = h * cw
            xv = x_ref[:, c0:c0 + cw]
            bv = jnp.max(xv, axis=0, keepdims=True)
            idx_local = jnp.min(
                jnp.where(xv == bv, rows, m_per), axis=0, keepdims=True
            )
            bi = (idx_local + my_pos * m_per).astype(jnp.float32)
            locals_.append((bv, bi))
            send_buf[h, 0:1, :] = bv
            send_buf[h, 1:2, :] = bi

            if h == 0:
                pl.semaphore_wait(barrier_sem, N_DEV - 1)

            for j in range(1, N_DEV):
                rdma = pltpu.make_async_remote_copy(
                    src_ref=send_buf.at[h],
                    dst_ref=part_buf.at[N_DEV - j, :, pl.ds(c0, cw)],
                    send_sem=send_sems.at[h, j - 1],
                    recv_sem=recv_sems.at[h, N_DEV - j],
                    device_id=(lax.rem(my_pos + j, N_DEV),),
                    device_id_type=pl.DeviceIdType.MESH,
                )
                rdma.start()
                rdmas.append((h, j, rdma))

        for h in range(H):
            c0 = h * cw
            bv, bi = locals_[h]
            for s in range(1, N_DEV):
                j = N_DEV - s
                next(r for hh, jj, r in rdmas if hh == h and jj == j).wait_recv()
                v = part_buf[s, 0:1, c0:c0 + cw]
                i = part_buf[s, 1:2, c0:c0 + cw]
                take = (v > bv) | ((v == bv) & (i < bi))
                bv = jnp.where(take, v, bv)
                bi = jnp.where(take, i, bi)
            out_ref[0:1, c0:c0 + cw] = bv
            out_ref[1:2, c0:c0 + cw] = bi

        for _, _, rdma in rdmas:
            rdma.wait_send()

    return pl.pallas_call(
        body,
        out_shape=jax.ShapeDtypeStruct((2, n), jnp.float32),
        in_specs=[pl.BlockSpec(memory_space=pltpu.VMEM)],
        out_specs=pl.BlockSpec(memory_space=pltpu.VMEM),
        scratch_shapes=[
            pltpu.VMEM((N_DEV, 2, n), jnp.float32),
            pltpu.VMEM((H, 2, cw), jnp.float32),
            pltpu.SemaphoreType.DMA((H, N_DEV - 1)),
            pltpu.SemaphoreType.DMA((H, N_DEV)),
        ],
        compiler_params=pltpu.CompilerParams(collective_id=0),
    )(x)


# device time: 8316 ns/iter; 1.0299x vs baseline; 1.0299x over previous
import jax
import jax.numpy as jnp
from jax import lax
from jax.experimental import pallas as pl
from jax.experimental.pallas import tpu as pltpu

N_DEV = 8


def kernel(x):
    m_per, n = x.shape

    def body(x_ref, out_ref, part_buf, send_sems, recv_sems):
        my_pos = lax.axis_index("i")

        barrier_sem = pltpu.get_barrier_semaphore()
        for j in range(1, N_DEV):
            peer = lax.rem(my_pos + j, N_DEV)
            pl.semaphore_signal(
                barrier_sem, inc=1,
                device_id=(peer,), device_id_type=pl.DeviceIdType.MESH,
            )

        xv = x_ref[:, :]
        bv = jnp.max(xv, axis=0, keepdims=True)
        rows = lax.broadcasted_iota(jnp.int32, (m_per, n), 0)
        idx_local = jnp.min(
            jnp.where(xv == bv, rows, m_per), axis=0, keepdims=True
        )
        bi = (idx_local + my_pos * m_per).astype(jnp.float32)
        part_buf[0, 0:1, :] = bv
        part_buf[0, 1:2, :] = bi

        pl.semaphore_wait(barrier_sem, N_DEV - 1)

        rdmas = []
        for j in range(1, N_DEV):
            rdma = pltpu.make_async_remote_copy(
                src_ref=part_buf.at[0],
                dst_ref=part_buf.at[N_DEV - j],
                send_sem=send_sems.at[j - 1],
                recv_sem=recv_sems.at[N_DEV - j],
                device_id=(lax.rem(my_pos + j, N_DEV),),
                device_id_type=pl.DeviceIdType.MESH,
            )
            rdma.start()
            rdmas.append(rdma)

        for s in range(1, N_DEV):
            rdmas[N_DEV - 1 - s].wait_recv()
            v = part_buf[s, 0:1, :]
            i = part_buf[s, 1:2, :]
            take = (v > bv) | ((v == bv) & (i < bi))
            bv = jnp.where(take, v, bv)
            bi = jnp.where(take, i, bi)

        out_ref[0:1, :] = bv
        out_ref[1:2, :] = bi

        for rdma in rdmas:
            rdma.wait_send()

    return pl.pallas_call(
        body,
        out_shape=jax.ShapeDtypeStruct((2, n), jnp.float32),
        in_specs=[pl.BlockSpec(memory_space=pltpu.VMEM)],
        out_specs=pl.BlockSpec(memory_space=pltpu.VMEM),
        scratch_shapes=[
            pltpu.VMEM((N_DEV, 2, n), jnp.float32),
            pltpu.SemaphoreType.DMA((N_DEV - 1,)),
            pltpu.SemaphoreType.DMA((N_DEV,)),
        ],
        compiler_params=pltpu.CompilerParams(collective_id=0),
    )(x)
